# Initial kernel scaffold; baseline (speedup 1.0000x reference)
import jax
import jax.numpy as jnp
from jax.experimental import pallas as pl


def kernel(feats, edge_index, W_in, b_in, W_hid, b_hid, W_out, b_out):
    raise NotImplementedError("write your pallas kernel here")


# speedup vs baseline: 8.3379x; 8.3379x over previous
import functools

import jax
import jax.numpy as jnp
from jax import lax
from jax.experimental import pallas as pl
from jax.experimental.pallas import tpu as pltpu
from jax.experimental.pallas import tpu_sc as plsc

_NC = 2
_NS = 16
_NW = _NC * _NS
_CH = 128


def _build_edge_pass(NP, D, K):
    R = NP // _NS
    T = K // 2
    mesh = plsc.VectorSubcoreMesh(
        core_axis_name="c", subcore_axis_name="s",
        num_cores=_NC, num_subcores=_NS)

    @functools.partial(
        pl.kernel,
        out_type=jax.ShapeDtypeStruct((_NC, NP, D), jnp.float32),
        mesh=mesh,
        compiler_params=pltpu.CompilerParams(use_tc_tiling_on_sc=False),
        scratch_types=[
            pltpu.VMEM((K, _CH), jnp.int32),
            pltpu.VMEM((K, _CH), jnp.int32),
            pltpu.VMEM((_CH, D), jnp.float32),
            pltpu.VMEM((_CH, D), jnp.float32),
            pltpu.VMEM((R, D), jnp.float32),
            pltpu.VMEM_SHARED((NP, D), jnp.float32),
            pltpu.SemaphoreType.DMA,
            pltpu.SemaphoreType.DMA,
            pltpu.SemaphoreType.DMA,
            pltpu.SemaphoreType.DMA,
        ],
    )
    def edge_pass(z_hbm, src_hbm, dst_hbm, out_hbm,
                  src_v, dst_v, rows0, rows1, bounce, acc,
                  gsem0, gsem1, ssem0, ssem1):
        c = lax.axis_index("c")
        s = lax.axis_index("s")
        wid = c * _NS + s

        pltpu.sync_copy(src_hbm.at[wid], src_v)
        pltpu.sync_copy(dst_hbm.at[wid], dst_v)

        zero16 = jnp.zeros((16,), jnp.float32)

        def zero_row(i, carry):
            for q in range(D // 16):
                bounce[i, pl.ds(q * 16, 16)] = zero16
            return carry

        lax.fori_loop(0, R, zero_row, 0)
        pltpu.sync_copy(bounce, acc.at[pl.ds(s * R, R)])
        plsc.subcore_barrier()

        pltpu.async_copy(z_hbm.at[src_v.at[0]], rows0, gsem0)

        def chunk_pair(t, carry):
            a = 2 * t
            b = a + 1
            pltpu.async_copy(z_hbm.at[src_v.at[b]], rows1, gsem1)
            pltpu.make_async_copy(z_hbm.at[src_v.at[a]], rows0, gsem0).wait()
            pltpu.async_copy(rows0, acc.at[dst_v.at[a]], ssem0, add=True)
            pltpu.make_async_copy(rows0, acc.at[dst_v.at[a]], ssem0).wait()

            @pl.when(t + 1 < T)
            def _prefetch():
                pltpu.async_copy(z_hbm.at[src_v.at[a + 2]], rows0, gsem0)

            pltpu.make_async_copy(z_hbm.at[src_v.at[b]], rows1, gsem1).wait()
            pltpu.async_copy(rows1, acc.at[dst_v.at[b]], ssem1, add=True)
            pltpu.make_async_copy(rows1, acc.at[dst_v.at[b]], ssem1).wait()
            return carry

        lax.fori_loop(0, T, chunk_pair, 0)
        plsc.subcore_barrier()

        pltpu.sync_copy(acc.at[pl.ds(s * R, R)], bounce)
        pltpu.sync_copy(bounce, out_hbm.at[c, pl.ds(s * R, R)])

    return edge_pass


def _build_deg_pass(NP, K):
    DW = 16
    R = NP // _NS
    mesh = plsc.VectorSubcoreMesh(
        core_axis_name="c", subcore_axis_name="s",
        num_cores=_NC, num_subcores=_NS)

    @functools.partial(
        pl.kernel,
        out_type=(jax.ShapeDtypeStruct((_NC, NP, DW), jnp.float32),
                  jax.ShapeDtypeStruct((_NC, NP, DW), jnp.float32)),
        mesh=mesh,
        compiler_params=pltpu.CompilerParams(use_tc_tiling_on_sc=False),
        scratch_types=[
            pltpu.VMEM((K, _CH), jnp.int32),
            pltpu.VMEM((K, _CH), jnp.int32),
            pltpu.VMEM((_CH, DW), jnp.float32),
            pltpu.VMEM((R, DW), jnp.float32),
            pltpu.VMEM_SHARED((NP, DW), jnp.float32),
            pltpu.VMEM_SHARED((NP, DW), jnp.float32),
            pltpu.SemaphoreType.DMA,
            pltpu.SemaphoreType.DMA,
        ],
    )
    def deg_pass(src_hbm, dst_hbm, outS_hbm, outD_hbm,
                 src_v, dst_v, ones, bounce, accS, accD, semS, semD):
        c = lax.axis_index("c")
        s = lax.axis_index("s")
        wid = c * _NS + s

        pltpu.sync_copy(src_hbm.at[wid], src_v)
        pltpu.sync_copy(dst_hbm.at[wid], dst_v)

        one16 = jnp.ones((16,), jnp.float32)
        zero16 = jnp.zeros((16,), jnp.float32)

        def fill_ones(i, carry):
            ones[i] = one16
            return carry

        lax.fori_loop(0, _CH, fill_ones, 0)

        def zero_row(i, carry):
            bounce[i] = zero16
            return carry

        lax.fori_loop(0, R, zero_row, 0)
        pltpu.sync_copy(bounce, accS.at[pl.ds(s * R, R)])
        pltpu.sync_copy(bounce, accD.at[pl.ds(s * R, R)])
        plsc.subcore_barrier()

        def chunk(t, carry):
            pltpu.async_copy(ones, accS.at[src_v.at[t]], semS, add=True)
            pltpu.async_copy(ones, accD.at[dst_v.at[t]], semD, add=True)
            pltpu.make_async_copy(ones, accS.at[src_v.at[t]], semS).wait()
            pltpu.make_async_copy(ones, accD.at[dst_v.at[t]], semD).wait()
            return carry

        lax.fori_loop(0, K, chunk, 0)
        plsc.subcore_barrier()

        pltpu.sync_copy(accS.at[pl.ds(s * R, R)], bounce)
        pltpu.sync_copy(bounce, outS_hbm.at[c, pl.ds(s * R, R)])
        pltpu.sync_copy(accD.at[pl.ds(s * R, R)], bounce)
        pltpu.sync_copy(bounce, outD_hbm.at[c, pl.ds(s * R, R)])

    return deg_pass


def kernel(feats, edge_index, W_in, b_in, W_hid, b_hid, W_out, b_out):
    N, d_in = feats.shape
    E = edge_index.shape[1]
    n_layers, d_h, _ = W_hid.shape
    d_out = W_out.shape[1]

    NP = -(-(N + 1) // 256) * 256
    EP = -(-E // (_NW * 2 * _CH)) * (_NW * 2 * _CH)
    K = EP // (_NW * _CH)

    pad = EP - E
    src = jnp.concatenate([edge_index[0], jnp.full((pad,), N, jnp.int32)])
    dst = jnp.concatenate([edge_index[1], jnp.full((pad,), N, jnp.int32)])
    src3 = src.reshape(_NW, K, _CH)
    dst3 = dst.reshape(_NW, K, _CH)

    b_in2 = b_in.reshape(1, d_h)
    b_hid2 = b_hid.reshape(n_layers, 1, d_h)
    b_out2 = b_out.reshape(1, d_out)

    deg_pass = _build_deg_pass(NP, K)
    edge32 = _build_edge_pass(NP, d_h, K)
    edge64 = _build_edge_pass(NP, d_out, K)

    degS, degD = deg_pass(src3, dst3)

    def tc_first(f_ref, w_ref, dS_ref, dD_ref, ns_ref, nd_ref, z_ref):
        dS = dS_ref[0, :, 0:1] + dS_ref[1, :, 0:1]
        dD = dD_ref[0, :, 0:1] + dD_ref[1, :, 0:1]
        ns = lax.rsqrt(jnp.maximum(dS, 1.0))
        nd = lax.rsqrt(jnp.maximum(dD, 1.0))
        ns_ref[...] = ns
        nd_ref[...] = nd
        z = jnp.dot(f_ref[...], w_ref[...], preferred_element_type=jnp.float32)
        z_ref[pl.ds(0, N), :] = z * ns[:N]
        z_ref[pl.ds(N, NP - N), :] = jnp.zeros((NP - N, d_h), jnp.float32)

    ns_arr, nd_arr, z = pl.pallas_call(
        tc_first,
        out_shape=(jax.ShapeDtypeStruct((NP, 1), jnp.float32),
                   jax.ShapeDtypeStruct((NP, 1), jnp.float32),
                   jax.ShapeDtypeStruct((NP, d_h), jnp.float32)),
    )(feats, W_in, degS, degD)

    def tc_layer(p_ref, nd_ref, ns_ref, b_ref, w_ref, h_ref, z_ref):
        agg = p_ref[0] + p_ref[1]
        h = jnp.maximum(agg * nd_ref[...] + b_ref[...], 0.0)
        h_ref[...] = h
        z_ref[...] = jnp.dot(h, w_ref[...],
                             preferred_element_type=jnp.float32) * ns_ref[...]

    tc_layer_call = pl.pallas_call(
        tc_layer,
        out_shape=(jax.ShapeDtypeStruct((NP, d_h), jnp.float32),
                   jax.ShapeDtypeStruct((NP, d_h), jnp.float32)),
    )

    n_convs = n_layers + 1
    b_stack = jnp.concatenate([b_in2[None], b_hid2], axis=0)
    w_stack = jnp.concatenate([W_hid, W_hid[n_layers - 1:]], axis=0)

    def conv_step(z_c, wb):
        b_i, w_i = wb
        part = edge32(z_c, src3, dst3)
        h, z_n = tc_layer_call(part, nd_arr, ns_arr, b_i, w_i)
        return z_n, h

    _, h_stack = lax.scan(conv_step, z, (b_stack, w_stack), length=n_convs)

    def tc_jk(h_ref, wout_ref, P_ref):
        hcat = jnp.concatenate([h_ref[i] for i in range(n_convs)], axis=1)
        P_ref[...] = jnp.dot(hcat, wout_ref[...],
                             preferred_element_type=jnp.float32)

    P = pl.pallas_call(
        tc_jk,
        out_shape=jax.ShapeDtypeStruct((NP, d_out), jnp.float32),
    )(h_stack, W_out)

    partF = edge64(P, src3, dst3)

    def tc_final(p_ref, b_ref, y_ref):
        p0 = p_ref[0]
        p1 = p_ref[1]
        y_ref[...] = p0[:N] + p1[:N] + b_ref[...]

    y = pl.pallas_call(
        tc_final,
        out_shape=jax.ShapeDtypeStruct((N, d_out), jnp.float32),
    )(partF, b_out2)
    return y
